# baseline (speedup 1.0000x reference)
import functools

import jax
import jax.numpy as jnp
from jax import lax
from jax.experimental import pallas as pl
from jax.experimental.pallas import tpu as pltpu
from jax.experimental.pallas import tpu_sc as plsc

N = 10000
E = 160000
NPAD = 10240
BR = 512
W_EDGE = 128
NC = 2
NS = 16
RPT = NPAD // NS
E_PADA = 163840
FW = 128
EPS = 1e-5
FN = float(N)


def _sc_mesh():
    return plsc.VectorSubcoreMesh(core_axis_name="c", subcore_axis_name="s")


def _sc_degree(dst_p):
    chunk = E_PADA // (NC * NS)
    nwin = chunk // W_EDGE

    @functools.partial(
        pl.kernel,
        out_type=jax.ShapeDtypeStruct((NC, NPAD), jnp.float32),
        mesh=_sc_mesh(),
        scratch_types=[
            pltpu.VMEM((chunk,), jnp.int32),
            pltpu.VMEM((W_EDGE,), jnp.float32),
            pltpu.VMEM((RPT,), jnp.float32),
            pltpu.VMEM_SHARED((NPAD,), jnp.float32),
            pltpu.SemaphoreType.DMA,
        ],
    )
    def deg_kernel(dst_hbm, out, dst_v, ones_v, dbounce, dacc, sem):
        c = lax.axis_index("c")
        s = lax.axis_index("s")
        for i in range(W_EDGE // 16):
            ones_v[pl.ds(16 * i, 16)] = jnp.ones((16,), jnp.float32)
        for i in range(RPT // 16):
            dbounce[pl.ds(16 * i, 16)] = jnp.zeros((16,), jnp.float32)
        pltpu.sync_copy(dbounce, dacc.at[pl.ds(s * RPT, RPT)])
        wid = c * NS + s
        pltpu.sync_copy(dst_hbm.at[pl.ds(wid * chunk, chunk)], dst_v)
        plsc.subcore_barrier()

        def win(w, carry):
            pltpu.sync_copy(
                ones_v, dacc.at[dst_v.at[pl.ds(w * W_EDGE, W_EDGE)]], add=True)
            return carry

        lax.fori_loop(0, nwin, win, 0)
        plsc.subcore_barrier()
        pltpu.sync_copy(dacc.at[pl.ds(s * RPT, RPT)], out.at[c, pl.ds(s * RPT, RPT)])

    return deg_kernel(dst_p)


def _sc_agg(src_p, dst_p, gtab, split_edges):
    chunk = E_PADA // (NC * NS) if split_edges else E_PADA // NS
    nwin = chunk // W_EDGE
    WG = 128
    NPH = 2
    pchunk = chunk // NPH
    pwin = nwin // NPH
    nit = pchunk // (2 * WG)

    @functools.partial(
        pl.kernel,
        out_type=jax.ShapeDtypeStruct((NC, NPAD, FW), jnp.float32),
        mesh=_sc_mesh(),
        scratch_types=[
            pltpu.VMEM((pchunk,), jnp.int32),
            pltpu.VMEM((pchunk,), jnp.int32),
            pltpu.VMEM((WG, FW), jnp.float32),
            pltpu.VMEM((WG, FW), jnp.float32),
            pltpu.VMEM_SHARED((NPAD, FW), jnp.float32),
            pltpu.SemaphoreType.DMA,
            pltpu.SemaphoreType.DMA,
        ],
    )
    def agg(src_hbm, dst_hbm, g_hbm, out, src_v, dst_v, buf_a, buf_b, acc,
            sem_a, sem_b):
        c = lax.axis_index("c")
        s = lax.axis_index("s")

        def run(tab):
            wid = (c * NS + s) if split_edges else s
            pltpu.sync_copy(tab.at[pl.ds(s * RPT, RPT)], acc.at[pl.ds(s * RPT, RPT)])
            plsc.subcore_barrier()

            def start(e, buf, sem):
                return pltpu.async_copy(tab.at[src_v.at[pl.ds(e, WG)]], buf, sem)

            def wait(buf, sem):
                pltpu.make_async_copy(tab.at[src_v.at[pl.ds(0, WG)]], buf, sem).wait()

            def scat(w, buf):
                pltpu.sync_copy(
                    buf, acc.at[dst_v.at[pl.ds(w * W_EDGE, W_EDGE)]], add=True)

            for ph in range(NPH):
                pltpu.sync_copy(
                    src_hbm.at[pl.ds(wid * chunk + ph * pchunk, pchunk)], src_v)
                pltpu.sync_copy(
                    dst_hbm.at[pl.ds(wid * chunk + ph * pchunk, pchunk)], dst_v)
                start(0, buf_a, sem_a)
                start(WG, buf_b, sem_b)

                def it(i, carry):
                    e0 = i * 2 * WG
                    wait(buf_a, sem_a)
                    scat(2 * i, buf_a)

                    @pl.when(i < nit - 1)
                    def _():
                        start(e0 + 2 * WG, buf_a, sem_a)

                    wait(buf_b, sem_b)
                    scat(2 * i + 1, buf_b)

                    @pl.when(i < nit - 1)
                    def _():
                        start(e0 + 3 * WG, buf_b, sem_b)

                    return carry

                lax.fori_loop(0, nit, it, 0)
            plsc.subcore_barrier()
            pltpu.sync_copy(
                acc.at[pl.ds(s * RPT, RPT)], out.at[c, pl.ds(s * RPT, RPT)])

        if split_edges:
            run(g_hbm)
        else:
            pl.when(c == 0)(lambda: run(g_hbm.at[0]))
            pl.when(c == 1)(lambda: run(g_hbm.at[1]))

    return agg(src_p, dst_p, gtab)


def _dis_block(d0_ref, d1_ref):
    return lax.rsqrt(d0_ref[...] + d1_ref[...] + 1.0)


def _mm_scale(xin, Wm, deg0, deg1, split):
    K = xin.shape[1]
    F = Wm.shape[1]
    if split:
        out_shape = jax.ShapeDtypeStruct((2, NPAD, F // 2), jnp.float32)
        out_spec = pl.BlockSpec((2, BR, F // 2), lambda i: (0, i, 0))
    else:
        out_shape = jax.ShapeDtypeStruct((NPAD, F), jnp.float32)
        out_spec = pl.BlockSpec((BR, F), lambda i: (i, 0))

    def body(x_ref, w_ref, d0_ref, d1_ref, o_ref):
        dis = _dis_block(d0_ref, d1_ref)
        p = jnp.dot(x_ref[...].astype(jnp.bfloat16), w_ref[...].astype(jnp.bfloat16),
                    preferred_element_type=jnp.float32)
        g = p * dis
        if split:
            o_ref[0] = g[:, : F // 2]
            o_ref[1] = g[:, F // 2 :]
        else:
            o_ref[...] = g

    return pl.pallas_call(
        body,
        grid=(NPAD // BR,),
        in_specs=[
            pl.BlockSpec((BR, K), lambda i: (i, 0)),
            pl.BlockSpec((K, F), lambda i: (0, 0)),
            pl.BlockSpec((BR, 1), lambda i: (i, 0)),
            pl.BlockSpec((BR, 1), lambda i: (i, 0)),
        ],
        out_specs=out_spec,
        out_shape=out_shape,
    )(xin, Wm, deg0, deg1)


def _bn_mm(A, deg0, deg1, brow, grow, berow, Wn, split):
    F = 2 * A.shape[2]
    Fn = Wn.shape[1]
    nb = NPAD // BR
    if split:
        out_shape = jax.ShapeDtypeStruct((2, NPAD, Fn // 2), jnp.float32)
        out_spec = pl.BlockSpec((2, BR, Fn // 2), lambda p, i: (0, i, 0))
    else:
        out_shape = jax.ShapeDtypeStruct((NPAD, Fn), jnp.float32)
        out_spec = pl.BlockSpec((BR, Fn), lambda p, i: (i, 0))

    def body(a_ref, d0_ref, d1_ref, b_ref, ga_ref, be_ref, w_ref, o_ref, st_ref):
        ph = pl.program_id(0)
        i = pl.program_id(1)
        dis = _dis_block(d0_ref, d1_ref)
        conv = jnp.concatenate([a_ref[0], a_ref[1]], axis=1) * dis + b_ref[...]
        rows = i * BR + lax.broadcasted_iota(jnp.int32, (BR, 1), 0)
        mask = rows < N

        @pl.when(ph == 0)
        def _():
            @pl.when(i == 0)
            def _():
                st_ref[...] = jnp.zeros((2, F), jnp.float32)

            s1 = jnp.sum(jnp.where(mask, conv, 0.0), axis=0, keepdims=True)
            s2 = jnp.sum(jnp.where(mask, conv * conv, 0.0), axis=0, keepdims=True)
            st_ref[...] += jnp.concatenate([s1, s2], axis=0)

        @pl.when(ph == 1)
        def _():
            m = st_ref[0:1, :] * (1.0 / FN)
            var = st_ref[1:2, :] * (1.0 / FN) - m * m
            y = ga_ref[...] * (conv - m) * lax.rsqrt(var + EPS) + be_ref[...]
            y = jnp.maximum(y, 0.0)
            y = jnp.where(mask, y, 0.0)
            p = jnp.dot(y.astype(jnp.bfloat16), w_ref[...].astype(jnp.bfloat16),
                        preferred_element_type=jnp.float32)
            g = p * dis
            if split:
                o_ref[0] = g[:, : Fn // 2]
                o_ref[1] = g[:, Fn // 2 :]
            else:
                o_ref[...] = g

    return pl.pallas_call(
        body,
        grid=(2, nb),
        in_specs=[
            pl.BlockSpec((2, BR, A.shape[2]), lambda p, i: (0, i, 0)),
            pl.BlockSpec((BR, 1), lambda p, i: (i, 0)),
            pl.BlockSpec((BR, 1), lambda p, i: (i, 0)),
            pl.BlockSpec((1, F), lambda p, i: (0, 0)),
            pl.BlockSpec((1, F), lambda p, i: (0, 0)),
            pl.BlockSpec((1, F), lambda p, i: (0, 0)),
            pl.BlockSpec((F, Fn), lambda p, i: (0, 0)),
        ],
        out_specs=out_spec,
        out_shape=out_shape,
        scratch_shapes=[pltpu.VMEM((2, F), jnp.float32)],
    )(A, deg0, deg1, brow, grow, berow, Wn)


def _final(P, g3, deg0, deg1, brow):
    F = P.shape[2]

    def body(p_ref, g3_ref, d0_ref, d1_ref, b_ref, o_ref):
        dis = _dis_block(d0_ref, d1_ref)
        o_ref[...] = (p_ref[0] + p_ref[1] - g3_ref[...]) * dis + b_ref[...]

    return pl.pallas_call(
        body,
        grid=(NPAD // BR,),
        in_specs=[
            pl.BlockSpec((2, BR, F), lambda i: (0, i, 0)),
            pl.BlockSpec((BR, F), lambda i: (i, 0)),
            pl.BlockSpec((BR, 1), lambda i: (i, 0)),
            pl.BlockSpec((BR, 1), lambda i: (i, 0)),
            pl.BlockSpec((1, F), lambda i: (0, 0)),
        ],
        out_specs=pl.BlockSpec((BR, F), lambda i: (i, 0)),
        out_shape=jax.ShapeDtypeStruct((N, F), jnp.float32),
    )(P, g3, deg0, deg1, brow)


def kernel(x, edge_index, W1, b1, g1, be1, W2, b2, g2, be2, W3, b3):
    src = edge_index[0]
    dst = edge_index[1]
    padidx = (jnp.arange(E_PADA - E, dtype=jnp.int32) % (NPAD - N)) + N
    src_p = jnp.concatenate([src, padidx])
    dst_p = jnp.concatenate([dst, padidx])
    xp = jnp.pad(x, ((0, NPAD - N), (0, 0)))

    degp = _sc_degree(dst_p)
    deg0 = degp[0].reshape(NPAD, 1)
    deg1 = degp[1].reshape(NPAD, 1)

    b1r, g1r, be1r = b1.reshape(1, -1), g1.reshape(1, -1), be1.reshape(1, -1)
    b2r, g2r, be2r = b2.reshape(1, -1), g2.reshape(1, -1), be2.reshape(1, -1)
    b3r = b3.reshape(1, -1)

    gt1 = _mm_scale(xp, W1, deg0, deg1, split=True)
    A1 = _sc_agg(src_p, dst_p, gt1, split_edges=False)
    gt2 = _bn_mm(A1, deg0, deg1, b1r, g1r, be1r, W2, split=True)
    A2 = _sc_agg(src_p, dst_p, gt2, split_edges=False)
    gt3 = _bn_mm(A2, deg0, deg1, b2r, g2r, be2r, W3, split=False)
    P3 = _sc_agg(src_p, dst_p, gt3, split_edges=True)
    return _final(P3, gt3, deg0, deg1, b3r)


# speedup vs baseline: 1.0064x; 1.0064x over previous
import functools

import jax
import jax.numpy as jnp
from jax import lax
from jax.experimental import pallas as pl
from jax.experimental.pallas import tpu as pltpu
from jax.experimental.pallas import tpu_sc as plsc

N = 10000
E = 160000
NPAD = 10240
BR = 512
W_EDGE = 128
NC = 2
NS = 16
RPT = NPAD // NS
E_PADA = 163840
FW = 128
EPS = 1e-5
FN = float(N)


def _sc_mesh():
    return plsc.VectorSubcoreMesh(core_axis_name="c", subcore_axis_name="s")


def _sc_degree(dst_p):
    chunk = E_PADA // (NC * NS)
    nwin = chunk // W_EDGE

    @functools.partial(
        pl.kernel,
        out_type=jax.ShapeDtypeStruct((NC, NPAD), jnp.float32),
        mesh=_sc_mesh(),
        scratch_types=[
            pltpu.VMEM((chunk,), jnp.int32),
            pltpu.VMEM((W_EDGE,), jnp.float32),
            pltpu.VMEM((RPT,), jnp.float32),
            pltpu.VMEM_SHARED((NPAD,), jnp.float32),
            pltpu.SemaphoreType.DMA,
        ],
    )
    def deg_kernel(dst_hbm, out, dst_v, ones_v, dbounce, dacc, sem):
        c = lax.axis_index("c")
        s = lax.axis_index("s")
        for i in range(W_EDGE // 16):
            ones_v[pl.ds(16 * i, 16)] = jnp.ones((16,), jnp.float32)
        for i in range(RPT // 16):
            dbounce[pl.ds(16 * i, 16)] = jnp.zeros((16,), jnp.float32)
        pltpu.sync_copy(dbounce, dacc.at[pl.ds(s * RPT, RPT)])
        wid = c * NS + s
        pltpu.sync_copy(dst_hbm.at[pl.ds(wid * chunk, chunk)], dst_v)
        plsc.subcore_barrier()

        def win(w, carry):
            pltpu.sync_copy(
                ones_v, dacc.at[dst_v.at[pl.ds(w * W_EDGE, W_EDGE)]], add=True)
            return carry

        lax.fori_loop(0, nwin, win, 0)
        plsc.subcore_barrier()
        pltpu.sync_copy(dacc.at[pl.ds(s * RPT, RPT)], out.at[c, pl.ds(s * RPT, RPT)])

    return deg_kernel(dst_p)


def _sc_agg(src_p, dst_p, gtab, split_edges):
    chunk = E_PADA // (NC * NS) if split_edges else E_PADA // NS
    nwin = chunk // W_EDGE
    WG = 128
    NPH = 2
    pchunk = chunk // NPH
    pwin = nwin // NPH
    nit = pchunk // (2 * WG)

    @functools.partial(
        pl.kernel,
        out_type=jax.ShapeDtypeStruct((NC, NPAD, FW), jnp.float32),
        mesh=_sc_mesh(),
        scratch_types=[
            pltpu.VMEM((pchunk,), jnp.int32),
            pltpu.VMEM((pchunk,), jnp.int32),
            pltpu.VMEM((WG, FW), jnp.float32),
            pltpu.VMEM((WG, FW), jnp.float32),
            pltpu.VMEM_SHARED((NPAD, FW), jnp.float32),
            pltpu.SemaphoreType.DMA,
            pltpu.SemaphoreType.DMA,
        ],
    )
    def agg(src_hbm, dst_hbm, g_hbm, out, src_v, dst_v, buf_a, buf_b, acc,
            sem_a, sem_b):
        c = lax.axis_index("c")
        s = lax.axis_index("s")

        def run(tab):
            wid = (c * NS + s) if split_edges else s
            pltpu.sync_copy(tab.at[pl.ds(s * RPT, RPT)], acc.at[pl.ds(s * RPT, RPT)])
            plsc.subcore_barrier()

            def start(e, buf, sem):
                return pltpu.async_copy(tab.at[src_v.at[pl.ds(e, WG)]], buf, sem)

            def wait(buf, sem):
                pltpu.make_async_copy(tab.at[src_v.at[pl.ds(0, WG)]], buf, sem).wait()

            def scat(w, buf):
                pltpu.sync_copy(
                    buf, acc.at[dst_v.at[pl.ds(w * W_EDGE, W_EDGE)]], add=True)

            for ph in range(NPH):
                pltpu.sync_copy(
                    src_hbm.at[pl.ds(wid * chunk + ph * pchunk, pchunk)], src_v)
                pltpu.sync_copy(
                    dst_hbm.at[pl.ds(wid * chunk + ph * pchunk, pchunk)], dst_v)
                start(0, buf_a, sem_a)
                start(WG, buf_b, sem_b)

                def it(i, carry):
                    e0 = i * 2 * WG
                    wait(buf_a, sem_a)
                    scat(2 * i, buf_a)

                    @pl.when(i < nit - 1)
                    def _():
                        start(e0 + 2 * WG, buf_a, sem_a)

                    wait(buf_b, sem_b)
                    scat(2 * i + 1, buf_b)

                    @pl.when(i < nit - 1)
                    def _():
                        start(e0 + 3 * WG, buf_b, sem_b)

                    return carry

                lax.fori_loop(0, nit, it, 0)
            plsc.subcore_barrier()
            pltpu.sync_copy(
                acc.at[pl.ds(s * RPT, RPT)], out.at[c, pl.ds(s * RPT, RPT)])

        if split_edges:
            run(g_hbm)
        else:
            pl.when(c == 0)(lambda: run(g_hbm.at[0]))
            pl.when(c == 1)(lambda: run(g_hbm.at[1]))

    return agg(src_p, dst_p, gtab)


def _dis_block(d0_ref, d1_ref):
    return lax.rsqrt(d0_ref[...] + d1_ref[...] + 1.0)


def _mm_scale(xin, Wm, deg0, deg1, split):
    K = xin.shape[1]
    F = Wm.shape[1]
    if split:
        out_shape = jax.ShapeDtypeStruct((2, NPAD, F // 2), jnp.float32)
        out_spec = pl.BlockSpec((2, BR, F // 2), lambda i: (0, i, 0))
    else:
        out_shape = jax.ShapeDtypeStruct((NPAD, F), jnp.float32)
        out_spec = pl.BlockSpec((BR, F), lambda i: (i, 0))

    def body(x_ref, w_ref, d0_ref, d1_ref, o_ref):
        dis = _dis_block(d0_ref, d1_ref)
        p = jnp.dot(x_ref[...].astype(jnp.bfloat16), w_ref[...].astype(jnp.bfloat16),
                    preferred_element_type=jnp.float32)
        g = p * dis
        if split:
            o_ref[0] = g[:, : F // 2]
            o_ref[1] = g[:, F // 2 :]
        else:
            o_ref[...] = g

    return pl.pallas_call(
        body,
        grid=(NPAD // BR,),
        in_specs=[
            pl.BlockSpec((BR, K), lambda i: (i, 0)),
            pl.BlockSpec((K, F), lambda i: (0, 0)),
            pl.BlockSpec((BR, 1), lambda i: (i, 0)),
            pl.BlockSpec((BR, 1), lambda i: (i, 0)),
        ],
        out_specs=out_spec,
        out_shape=out_shape,
    )(xin, Wm, deg0, deg1)


def _bn_mm(A, deg0, deg1, brow, grow, berow, Wn, split):
    F = 2 * A.shape[2]
    Fn = Wn.shape[1]
    nb = NPAD // BR
    if split:
        out_shape = jax.ShapeDtypeStruct((2, NPAD, Fn // 2), jnp.float32)
        out_spec = pl.BlockSpec((2, BR, Fn // 2), lambda p, i: (0, i * p, 0))
    else:
        out_shape = jax.ShapeDtypeStruct((NPAD, Fn), jnp.float32)
        out_spec = pl.BlockSpec((BR, Fn), lambda p, i: (i * p, 0))

    def body(a_ref, d0_ref, d1_ref, b_ref, ga_ref, be_ref, w_ref, o_ref, st_ref):
        ph = pl.program_id(0)
        i = pl.program_id(1)
        dis = _dis_block(d0_ref, d1_ref)
        conv = jnp.concatenate([a_ref[0], a_ref[1]], axis=1) * dis + b_ref[...]
        rows = i * BR + lax.broadcasted_iota(jnp.int32, (BR, 1), 0)
        mask = rows < N

        @pl.when(ph == 0)
        def _():
            @pl.when(i == 0)
            def _():
                st_ref[...] = jnp.zeros((2, F), jnp.float32)

            s1 = jnp.sum(jnp.where(mask, conv, 0.0), axis=0, keepdims=True)
            s2 = jnp.sum(jnp.where(mask, conv * conv, 0.0), axis=0, keepdims=True)
            st_ref[...] += jnp.concatenate([s1, s2], axis=0)

        @pl.when(ph == 1)
        def _():
            m = st_ref[0:1, :] * (1.0 / FN)
            var = st_ref[1:2, :] * (1.0 / FN) - m * m
            y = ga_ref[...] * (conv - m) * lax.rsqrt(var + EPS) + be_ref[...]
            y = jnp.maximum(y, 0.0)
            y = jnp.where(mask, y, 0.0)
            p = jnp.dot(y.astype(jnp.bfloat16), w_ref[...].astype(jnp.bfloat16),
                        preferred_element_type=jnp.float32)
            g = p * dis
            if split:
                o_ref[0] = g[:, : Fn // 2]
                o_ref[1] = g[:, Fn // 2 :]
            else:
                o_ref[...] = g

    return pl.pallas_call(
        body,
        grid=(2, nb),
        in_specs=[
            pl.BlockSpec((2, BR, A.shape[2]), lambda p, i: (0, i, 0)),
            pl.BlockSpec((BR, 1), lambda p, i: (i, 0)),
            pl.BlockSpec((BR, 1), lambda p, i: (i, 0)),
            pl.BlockSpec((1, F), lambda p, i: (0, 0)),
            pl.BlockSpec((1, F), lambda p, i: (0, 0)),
            pl.BlockSpec((1, F), lambda p, i: (0, 0)),
            pl.BlockSpec((F, Fn), lambda p, i: (0, 0)),
        ],
        out_specs=out_spec,
        out_shape=out_shape,
        scratch_shapes=[pltpu.VMEM((2, F), jnp.float32)],
    )(A, deg0, deg1, brow, grow, berow, Wn)


def _final(P, g3, deg0, deg1, brow):
    F = P.shape[2]

    def body(p_ref, g3_ref, d0_ref, d1_ref, b_ref, o_ref):
        dis = _dis_block(d0_ref, d1_ref)
        o_ref[...] = (p_ref[0] + p_ref[1] - g3_ref[...]) * dis + b_ref[...]

    return pl.pallas_call(
        body,
        grid=(NPAD // BR,),
        in_specs=[
            pl.BlockSpec((2, BR, F), lambda i: (0, i, 0)),
            pl.BlockSpec((BR, F), lambda i: (i, 0)),
            pl.BlockSpec((BR, 1), lambda i: (i, 0)),
            pl.BlockSpec((BR, 1), lambda i: (i, 0)),
            pl.BlockSpec((1, F), lambda i: (0, 0)),
        ],
        out_specs=pl.BlockSpec((BR, F), lambda i: (i, 0)),
        out_shape=jax.ShapeDtypeStruct((N, F), jnp.float32),
    )(P, g3, deg0, deg1, brow)


def kernel(x, edge_index, W1, b1, g1, be1, W2, b2, g2, be2, W3, b3):
    src = edge_index[0]
    dst = edge_index[1]
    padidx = (jnp.arange(E_PADA - E, dtype=jnp.int32) % (NPAD - N)) + N
    src_p = jnp.concatenate([src, padidx])
    dst_p = jnp.concatenate([dst, padidx])
    xp = jnp.pad(x, ((0, NPAD - N), (0, 0)))

    degp = _sc_degree(dst_p)
    deg0 = degp[0].reshape(NPAD, 1)
    deg1 = degp[1].reshape(NPAD, 1)

    b1r, g1r, be1r = b1.reshape(1, -1), g1.reshape(1, -1), be1.reshape(1, -1)
    b2r, g2r, be2r = b2.reshape(1, -1), g2.reshape(1, -1), be2.reshape(1, -1)
    b3r = b3.reshape(1, -1)

    gt1 = _mm_scale(xp, W1, deg0, deg1, split=True)
    A1 = _sc_agg(src_p, dst_p, gt1, split_edges=False)
    gt2 = _bn_mm(A1, deg0, deg1, b1r, g1r, be1r, W2, split=True)
    A2 = _sc_agg(src_p, dst_p, gt2, split_edges=False)
    gt3 = _bn_mm(A2, deg0, deg1, b2r, g2r, be2r, W3, split=False)
    P3 = _sc_agg(src_p, dst_p, gt3, split_edges=True)
    return _final(P3, gt3, deg0, deg1, b3r)


# speedup vs baseline: 1.0114x; 1.0049x over previous
import functools

import jax
import jax.numpy as jnp
from jax import lax
from jax.experimental import pallas as pl
from jax.experimental.pallas import tpu as pltpu
from jax.experimental.pallas import tpu_sc as plsc

N = 10000
E = 160000
NPAD = 10240
BR = 512
W_EDGE = 128
NC = 2
NS = 16
RPT = NPAD // NS
E_PADA = 163840
FW = 128
EPS = 1e-5
FN = float(N)


def _sc_mesh():
    return plsc.VectorSubcoreMesh(core_axis_name="c", subcore_axis_name="s")


def _sc_degree(dst_p):
    chunk = E_PADA // (NC * NS)
    nwin = chunk // W_EDGE

    @functools.partial(
        pl.kernel,
        out_type=jax.ShapeDtypeStruct((NC, NPAD), jnp.float32),
        mesh=_sc_mesh(),
        scratch_types=[
            pltpu.VMEM((chunk,), jnp.int32),
            pltpu.VMEM((W_EDGE,), jnp.float32),
            pltpu.VMEM((RPT,), jnp.float32),
            pltpu.VMEM_SHARED((NPAD,), jnp.float32),
            pltpu.SemaphoreType.DMA,
        ],
    )
    def deg_kernel(dst_hbm, out, dst_v, ones_v, dbounce, dacc, sem):
        c = lax.axis_index("c")
        s = lax.axis_index("s")
        for i in range(W_EDGE // 16):
            ones_v[pl.ds(16 * i, 16)] = jnp.ones((16,), jnp.float32)
        for i in range(RPT // 16):
            dbounce[pl.ds(16 * i, 16)] = jnp.zeros((16,), jnp.float32)
        pltpu.sync_copy(dbounce, dacc.at[pl.ds(s * RPT, RPT)])
        wid = c * NS + s
        pltpu.sync_copy(dst_hbm.at[pl.ds(wid * chunk, chunk)], dst_v)
        plsc.subcore_barrier()

        def win(w, carry):
            pltpu.sync_copy(
                ones_v, dacc.at[dst_v.at[pl.ds(w * W_EDGE, W_EDGE)]], add=True)
            return carry

        lax.fori_loop(0, nwin, win, 0)
        plsc.subcore_barrier()
        pltpu.sync_copy(dacc.at[pl.ds(s * RPT, RPT)], out.at[c, pl.ds(s * RPT, RPT)])

    return deg_kernel(dst_p)


def _sc_agg(src_p, dst_p, gtab, split_edges):
    chunk = E_PADA // (NC * NS) if split_edges else E_PADA // NS
    nwin = chunk // W_EDGE
    WG = 128
    NPH = 2
    pchunk = chunk // NPH
    nit = pchunk // (2 * WG)

    @functools.partial(
        pl.kernel,
        out_type=jax.ShapeDtypeStruct((NC, NPAD, FW), jnp.float32),
        mesh=_sc_mesh(),
        scratch_types=[
            pltpu.VMEM((pchunk,), jnp.int32),
            pltpu.VMEM((pchunk,), jnp.int32),
            pltpu.VMEM((WG, FW), jnp.float32),
            pltpu.VMEM((WG, FW), jnp.float32),
            pltpu.VMEM_SHARED((NPAD, FW), jnp.float32),
            pltpu.SemaphoreType.DMA,
            pltpu.SemaphoreType.DMA,
        ],
    )
    def agg(src_hbm, dst_hbm, g_hbm, out, src_v, dst_v, buf_a, buf_b, acc,
            sem_a, sem_b):
        c = lax.axis_index("c")
        s = lax.axis_index("s")

        def run(tab):
            wid = (c * NS + s) if split_edges else s
            pltpu.sync_copy(tab.at[pl.ds(s * RPT, RPT)], acc.at[pl.ds(s * RPT, RPT)])
            plsc.subcore_barrier()

            def start(e, buf, sem):
                return pltpu.async_copy(tab.at[src_v.at[pl.ds(e, WG)]], buf, sem)

            def wait(buf, sem):
                pltpu.make_async_copy(tab.at[src_v.at[pl.ds(0, WG)]], buf, sem).wait()

            def scat(w, buf):
                pltpu.sync_copy(
                    buf, acc.at[dst_v.at[pl.ds(w * W_EDGE, W_EDGE)]], add=True)

            for ph in range(NPH):
                pltpu.sync_copy(
                    src_hbm.at[pl.ds(wid * chunk + ph * pchunk, pchunk)], src_v)
                pltpu.sync_copy(
                    dst_hbm.at[pl.ds(wid * chunk + ph * pchunk, pchunk)], dst_v)
                start(0, buf_a, sem_a)
                start(WG, buf_b, sem_b)

                def it(i, carry):
                    e0 = i * 2 * WG
                    wait(buf_a, sem_a)
                    scat(2 * i, buf_a)

                    @pl.when(i < nit - 1)
                    def _():
                        start(e0 + 2 * WG, buf_a, sem_a)

                    wait(buf_b, sem_b)
                    scat(2 * i + 1, buf_b)

                    @pl.when(i < nit - 1)
                    def _():
                        start(e0 + 3 * WG, buf_b, sem_b)

                    return carry

                lax.fori_loop(0, nit, it, 0)
            plsc.subcore_barrier()
            pltpu.sync_copy(
                acc.at[pl.ds(s * RPT, RPT)], out.at[c, pl.ds(s * RPT, RPT)])

        if split_edges:
            run(g_hbm)
        else:
            pl.when(c == 0)(lambda: run(g_hbm.at[0]))
            pl.when(c == 1)(lambda: run(g_hbm.at[1]))

    return agg(src_p, dst_p, gtab)


def _dis_block(d0_ref, d1_ref):
    return lax.rsqrt(d0_ref[...] + d1_ref[...] + 1.0)


def _mm_scale(xin, Wm, deg0, deg1, split):
    K = xin.shape[1]
    F = Wm.shape[1]
    if split:
        out_shape = jax.ShapeDtypeStruct((2, NPAD, F // 2), jnp.float32)
        out_spec = pl.BlockSpec((2, BR, F // 2), lambda i: (0, i, 0))
    else:
        out_shape = jax.ShapeDtypeStruct((NPAD, F), jnp.float32)
        out_spec = pl.BlockSpec((BR, F), lambda i: (i, 0))

    def body(x_ref, w_ref, d0_ref, d1_ref, o_ref):
        dis = _dis_block(d0_ref, d1_ref)
        p = jnp.dot(x_ref[...].astype(jnp.bfloat16), w_ref[...].astype(jnp.bfloat16),
                    preferred_element_type=jnp.float32)
        g = p * dis
        if split:
            o_ref[0] = g[:, : F // 2]
            o_ref[1] = g[:, F // 2 :]
        else:
            o_ref[...] = g

    return pl.pallas_call(
        body,
        grid=(NPAD // BR,),
        in_specs=[
            pl.BlockSpec((BR, K), lambda i: (i, 0)),
            pl.BlockSpec((K, F), lambda i: (0, 0)),
            pl.BlockSpec((BR, 1), lambda i: (i, 0)),
            pl.BlockSpec((BR, 1), lambda i: (i, 0)),
        ],
        out_specs=out_spec,
        out_shape=out_shape,
    )(xin, Wm, deg0, deg1)


def _bn_mm(A, deg0, deg1, brow, grow, berow, Wn, split):
    F = 2 * A.shape[2]
    Fn = Wn.shape[1]
    nb = NPAD // BR
    if split:
        out_shape = jax.ShapeDtypeStruct((2, NPAD, Fn // 2), jnp.float32)
        out_spec = pl.BlockSpec((2, BR, Fn // 2), lambda p, i: (0, i * p, 0))
    else:
        out_shape = jax.ShapeDtypeStruct((NPAD, Fn), jnp.float32)
        out_spec = pl.BlockSpec((BR, Fn), lambda p, i: (i * p, 0))

    def body(a_ref, d0_ref, d1_ref, b_ref, ga_ref, be_ref, w_ref, o_ref, st_ref):
        ph = pl.program_id(0)
        i = pl.program_id(1)
        dis = _dis_block(d0_ref, d1_ref)
        conv = jnp.concatenate([a_ref[0], a_ref[1]], axis=1) * dis + b_ref[...]
        rows = i * BR + lax.broadcasted_iota(jnp.int32, (BR, 1), 0)
        mask = rows < N

        @pl.when(ph == 0)
        def _():
            @pl.when(i == 0)
            def _():
                st_ref[...] = jnp.zeros((2, F), jnp.float32)

            s1 = jnp.sum(jnp.where(mask, conv, 0.0), axis=0, keepdims=True)
            s2 = jnp.sum(jnp.where(mask, conv * conv, 0.0), axis=0, keepdims=True)
            st_ref[...] += jnp.concatenate([s1, s2], axis=0)

        @pl.when(ph == 1)
        def _():
            m = st_ref[0:1, :] * (1.0 / FN)
            var = st_ref[1:2, :] * (1.0 / FN) - m * m
            y = ga_ref[...] * (conv - m) * lax.rsqrt(var + EPS) + be_ref[...]
            y = jnp.maximum(y, 0.0)
            y = jnp.where(mask, y, 0.0)
            p = jnp.dot(y.astype(jnp.bfloat16), w_ref[...].astype(jnp.bfloat16),
                        preferred_element_type=jnp.float32)
            g = p * dis
            if split:
                o_ref[0] = g[:, : Fn // 2]
                o_ref[1] = g[:, Fn // 2 :]
            else:
                o_ref[...] = g

    return pl.pallas_call(
        body,
        grid=(2, nb),
        in_specs=[
            pl.BlockSpec((2, BR, A.shape[2]), lambda p, i: (0, i, 0)),
            pl.BlockSpec((BR, 1), lambda p, i: (i, 0)),
            pl.BlockSpec((BR, 1), lambda p, i: (i, 0)),
            pl.BlockSpec((1, F), lambda p, i: (0, 0)),
            pl.BlockSpec((1, F), lambda p, i: (0, 0)),
            pl.BlockSpec((1, F), lambda p, i: (0, 0)),
            pl.BlockSpec((F, Fn), lambda p, i: (0, 0)),
        ],
        out_specs=out_spec,
        out_shape=out_shape,
        scratch_shapes=[pltpu.VMEM((2, F), jnp.float32)],
    )(A, deg0, deg1, brow, grow, berow, Wn)


def _final(P, g3, deg0, deg1, brow):
    F = P.shape[2]

    def body(p_ref, g3_ref, d0_ref, d1_ref, b_ref, o_ref):
        dis = _dis_block(d0_ref, d1_ref)
        o_ref[...] = (p_ref[0] + p_ref[1] - g3_ref[...]) * dis + b_ref[...]

    return pl.pallas_call(
        body,
        grid=(NPAD // BR,),
        in_specs=[
            pl.BlockSpec((2, BR, F), lambda i: (0, i, 0)),
            pl.BlockSpec((BR, F), lambda i: (i, 0)),
            pl.BlockSpec((BR, 1), lambda i: (i, 0)),
            pl.BlockSpec((BR, 1), lambda i: (i, 0)),
            pl.BlockSpec((1, F), lambda i: (0, 0)),
        ],
        out_specs=pl.BlockSpec((BR, F), lambda i: (i, 0)),
        out_shape=jax.ShapeDtypeStruct((N, F), jnp.float32),
    )(P, g3, deg0, deg1, brow)


def kernel(x, edge_index, W1, b1, g1, be1, W2, b2, g2, be2, W3, b3):
    src = edge_index[0]
    dst = edge_index[1]
    padidx = (jnp.arange(E_PADA - E, dtype=jnp.int32) % (NPAD - N)) + N
    src_p = jnp.concatenate([src, padidx])
    dst_p = jnp.concatenate([dst, padidx])
    xp = jnp.pad(x, ((0, NPAD - N), (0, 0)))

    degp = _sc_degree(dst_p)
    deg0 = degp[0].reshape(NPAD, 1)
    deg1 = degp[1].reshape(NPAD, 1)

    b1r, g1r, be1r = b1.reshape(1, -1), g1.reshape(1, -1), be1.reshape(1, -1)
    b2r, g2r, be2r = b2.reshape(1, -1), g2.reshape(1, -1), be2.reshape(1, -1)
    b3r = b3.reshape(1, -1)

    gt1 = _mm_scale(xp, W1, deg0, deg1, split=True)
    A1 = _sc_agg(src_p, dst_p, gt1, split_edges=False)
    gt2 = _bn_mm(A1, deg0, deg1, b1r, g1r, be1r, W2, split=True)
    A2 = _sc_agg(src_p, dst_p, gt2, split_edges=False)
    gt3 = _bn_mm(A2, deg0, deg1, b2r, g2r, be2r, W3, split=False)
    P3 = _sc_agg(src_p, dst_p, gt3, split_edges=True)
    return _final(P3, gt3, deg0, deg1, b3r)
